# Initial kernel scaffold; baseline (speedup 1.0000x reference)
import functools

import jax
import jax.numpy as jnp
from jax import lax
from jax.experimental import pallas as pl
from jax.experimental.pallas import tpu as pltpu
from jax.experimental.pallas import tpu_sc as plsc

N = 10000
E = 320000
D = 128
DO = 8

N_PAD = 10240
ROWS_PER_TILE = N_PAD // 16
CHUNK = 128
CHUNKS_PER_TILE = 79
EPT = CHUNKS_PER_TILE * CHUNK
E_PAD = EPT * 32
ROW_BLOCKS = E_PAD // CHUNK

_mesh = plsc.VectorSubcoreMesh(core_axis_name="c", subcore_axis_name="s")


@functools.partial(
    pl.kernel,
    out_type=jax.ShapeDtypeStruct((2, N_PAD), jnp.float32),
    mesh=_mesh,
    scratch_types=[
        pltpu.VMEM((CHUNKS_PER_TILE, CHUNK), jnp.int32),
        pltpu.VMEM((CHUNK,), jnp.float32),
        pltpu.VMEM_SHARED((N_PAD,), jnp.float32),
    ],
)
def _sc_degree(dst_hbm, ones_hbm, z1_hbm, deg_hbm, dst_v, ones_v, acc):
    c = lax.axis_index("c")
    s = lax.axis_index("s")
    wid = c * 16 + s
    pltpu.sync_copy(dst_hbm.at[pl.ds(wid * CHUNKS_PER_TILE, CHUNKS_PER_TILE)], dst_v)
    pltpu.sync_copy(ones_hbm, ones_v)
    pltpu.sync_copy(z1_hbm, acc.at[pl.ds(s * ROWS_PER_TILE, ROWS_PER_TILE)])
    plsc.subcore_barrier()
    for i in range(CHUNKS_PER_TILE):
        pltpu.sync_copy(ones_v, acc.at[dst_v.at[i]], add=True)
    plsc.subcore_barrier()
    rb = s * ROWS_PER_TILE
    pltpu.sync_copy(acc.at[pl.ds(rb, ROWS_PER_TILE)], deg_hbm.at[c, pl.ds(rb, ROWS_PER_TILE)])


def _make_sc_aggregate(d):
    @functools.partial(
        pl.kernel,
        out_type=jax.ShapeDtypeStruct((2, N_PAD, d), jnp.float32),
        mesh=_mesh,
        scratch_types=[
            pltpu.VMEM((CHUNKS_PER_TILE, CHUNK), jnp.int32),
            pltpu.VMEM((CHUNKS_PER_TILE, CHUNK), jnp.int32),
            pltpu.VMEM((CHUNK, d), jnp.float32),
            pltpu.VMEM((CHUNK, d), jnp.float32),
            pltpu.VMEM_SHARED((N_PAD, d), jnp.float32),
            pltpu.SemaphoreType.DMA,
            pltpu.SemaphoreType.DMA,
        ],
    )
    def agg(g_hbm, src_hbm, dst_hbm, z_hbm, out_hbm,
            src_v, dst_v, buf_a, buf_b, acc, sem_a, sem_b):
        c = lax.axis_index("c")
        s = lax.axis_index("s")
        wid = c * 16 + s
        base = wid * CHUNKS_PER_TILE
        pltpu.sync_copy(src_hbm.at[pl.ds(base, CHUNKS_PER_TILE)], src_v)
        pltpu.sync_copy(dst_hbm.at[pl.ds(base, CHUNKS_PER_TILE)], dst_v)
        rb = s * ROWS_PER_TILE
        pltpu.sync_copy(z_hbm, acc.at[pl.ds(rb, ROWS_PER_TILE)])
        plsc.subcore_barrier()

        bufs = (buf_a, buf_b)
        sems = (sem_a, sem_b)
        descs = [None, None]
        descs[0] = pltpu.async_copy(g_hbm.at[src_v.at[0]], buf_a, sem_a)
        for i in range(CHUNKS_PER_TILE):
            cur = i % 2
            nxt = (i + 1) % 2
            if i + 1 < CHUNKS_PER_TILE:
                descs[nxt] = pltpu.async_copy(
                    g_hbm.at[src_v.at[i + 1]], bufs[nxt], sems[nxt])
            descs[cur].wait()
            pltpu.sync_copy(bufs[cur], acc.at[dst_v.at[i]], add=True)
        plsc.subcore_barrier()
        for k in range(ROWS_PER_TILE // CHUNK):
            pltpu.sync_copy(acc.at[pl.ds(rb + k * CHUNK, CHUNK)],
                            out_hbm.at[c, pl.ds(rb + k * CHUNK, CHUNK)])

    return agg


_sc_agg_wide = _make_sc_aggregate(D)
_sc_agg_narrow = _make_sc_aggregate(DO)


_R = 1024
_GRID = N_PAD // _R


def _tc1_body(x_ref, fi_ref, w1_ref, d0_ref, d1_ref, g1_ref, s_ref):
    deg = d0_ref[...] + d1_ref[...] + 1.0
    s = lax.rsqrt(deg)
    xw = x_ref[...] * jax.nn.sigmoid(fi_ref[...])
    h = jnp.dot(xw, w1_ref[...], preferred_element_type=jnp.float32)
    g1_ref[...] = h * s
    s_ref[...] = s


def _tc1(x_p, fi, w1, d0, d1):
    return pl.pallas_call(
        _tc1_body,
        grid=(_GRID,),
        in_specs=[
            pl.BlockSpec((_R, D), lambda i: (i, 0)),
            pl.BlockSpec((1, D), lambda i: (0, 0)),
            pl.BlockSpec((D, D), lambda i: (0, 0)),
            pl.BlockSpec((_R, 1), lambda i: (i, 0)),
            pl.BlockSpec((_R, 1), lambda i: (i, 0)),
        ],
        out_specs=[
            pl.BlockSpec((_R, D), lambda i: (i, 0)),
            pl.BlockSpec((_R, 1), lambda i: (i, 0)),
        ],
        out_shape=[
            jax.ShapeDtypeStruct((N_PAD, D), jnp.float32),
            jax.ShapeDtypeStruct((N_PAD, 1), jnp.float32),
        ],
    )(x_p, fi, w1, d0, d1)


def _tc2_body(p0_ref, p1_ref, g1_ref, s_ref, b1_ref, w2_ref, g2_ref):
    s = s_ref[...]
    z = s * (p0_ref[...] + p1_ref[...] + g1_ref[...]) + b1_ref[...]
    h = jnp.maximum(z, 0.0)
    p = jnp.dot(h, w2_ref[...], preferred_element_type=jnp.float32)
    g2_ref[...] = p * s


def _tc2(p0, p1, g1, s, b1, w2p):
    return pl.pallas_call(
        _tc2_body,
        grid=(_GRID,),
        in_specs=[
            pl.BlockSpec((_R, D), lambda i: (i, 0)),
            pl.BlockSpec((_R, D), lambda i: (i, 0)),
            pl.BlockSpec((_R, D), lambda i: (i, 0)),
            pl.BlockSpec((_R, 1), lambda i: (i, 0)),
            pl.BlockSpec((1, D), lambda i: (0, 0)),
            pl.BlockSpec((D, DO), lambda i: (0, 0)),
        ],
        out_specs=pl.BlockSpec((_R, DO), lambda i: (i, 0)),
        out_shape=jax.ShapeDtypeStruct((N_PAD, DO), jnp.float32),
    )(p0, p1, g1, s, b1, w2p)


def _tc3_body(q0_ref, q1_ref, g2_ref, s_ref, b2_ref, o_ref):
    o_ref[...] = s_ref[...] * (q0_ref[...] + q1_ref[...] + g2_ref[...]) + b2_ref[...]


def _tc3(q0, q1, g2, s, b2p):
    return pl.pallas_call(
        _tc3_body,
        grid=(_GRID,),
        in_specs=[
            pl.BlockSpec((_R, DO), lambda i: (i, 0)),
            pl.BlockSpec((_R, DO), lambda i: (i, 0)),
            pl.BlockSpec((_R, DO), lambda i: (i, 0)),
            pl.BlockSpec((_R, 1), lambda i: (i, 0)),
            pl.BlockSpec((1, DO), lambda i: (0, 0)),
        ],
        out_specs=pl.BlockSpec((_R, DO), lambda i: (i, 0)),
        out_shape=jax.ShapeDtypeStruct((N_PAD, DO), jnp.float32),
    )(q0, q1, g2, s, b2p)


def kernel(x, edge_index, feature_importance, W1, b1, W2, b2):
    f32 = jnp.float32
    pad_e = E_PAD - E
    src = jnp.concatenate([edge_index[0], jnp.full((pad_e,), N, jnp.int32)])
    dst = jnp.concatenate([edge_index[1], jnp.full((pad_e,), N, jnp.int32)])
    src2d = src.reshape(ROW_BLOCKS, CHUNK)
    dst2d = dst.reshape(ROW_BLOCKS, CHUNK)

    x_p = jnp.zeros((N_PAD, D), f32).at[:N].set(x)
    fi = feature_importance.reshape(1, D)
    b1r = b1.reshape(1, D)
    w2p = jnp.zeros((D, DO), f32).at[:, : W2.shape[1]].set(W2)
    b2p = jnp.zeros((1, DO), f32).at[0, : b2.shape[0]].set(b2)

    ones_c = jnp.ones((CHUNK,), f32)
    z1 = jnp.zeros((ROWS_PER_TILE,), f32)
    z_wide = jnp.zeros((ROWS_PER_TILE, D), f32)
    z_narrow = jnp.zeros((ROWS_PER_TILE, DO), f32)

    deg_parts = _sc_degree(dst2d, ones_c, z1)
    d0 = deg_parts[0].reshape(N_PAD, 1)
    d1 = deg_parts[1].reshape(N_PAD, 1)

    g1, s = _tc1(x_p, fi, W1, d0, d1)

    agg1 = _sc_agg_wide(g1, src2d, dst2d, z_wide)

    g2 = _tc2(agg1[0], agg1[1], g1, s, b1r, w2p)

    agg2 = _sc_agg_narrow(g2, src2d, dst2d, z_narrow)

    out = _tc3(agg2[0], agg2[1], g2, s, b2p)
    return out[:N, : W2.shape[1]]


# speedup vs baseline: 12.8124x; 12.8124x over previous
import functools

import jax
import jax.numpy as jnp
from jax import lax
from jax.experimental import pallas as pl
from jax.experimental.pallas import tpu as pltpu
from jax.experimental.pallas import tpu_sc as plsc

N = 10000
E = 320000
D = 128
DH = 64
DO = 8

N_PAD = 10240
ROWS_PER_TILE = N_PAD // 16
CHUNK = 128
CHUNKS_PER_TILE = 80
EPT = CHUNKS_PER_TILE * CHUNK
E_PAD = EPT * 32
ROW_BLOCKS = E_PAD // CHUNK

SHIFT = 14
MASK = (1 << SHIFT) - 1

_mesh = plsc.VectorSubcoreMesh(core_axis_name="c", subcore_axis_name="s")


def _unpack_edges(packed_v, src_v, dst_v):
    def body(i, _):
        for k in range(CHUNK // 16):
            v = packed_v[i, pl.ds(k * 16, 16)]
            src_v[i, pl.ds(k * 16, 16)] = v & MASK
            dst_v[i, pl.ds(k * 16, 16)] = lax.shift_right_logical(v, SHIFT)
        return 0

    lax.fori_loop(0, CHUNKS_PER_TILE, body, 0)


@functools.partial(
    pl.kernel,
    out_type=jax.ShapeDtypeStruct((2 * N_PAD,), jnp.float32),
    mesh=_mesh,
    compiler_params=pltpu.CompilerParams(use_tc_tiling_on_sc=False),
    scratch_types=[
        pltpu.VMEM((CHUNKS_PER_TILE, CHUNK), jnp.int32),
        pltpu.VMEM((CHUNKS_PER_TILE, CHUNK), jnp.int32),
        pltpu.VMEM((CHUNKS_PER_TILE, CHUNK), jnp.int32),
        pltpu.VMEM((CHUNK,), jnp.float32),
        pltpu.VMEM_SHARED((N_PAD,), jnp.float32),
    ],
)
def _sc_degree(pk_hbm, ones_hbm, z1_hbm, deg_hbm, pk_v, src_v, dst_v, ones_v, acc):
    c = lax.axis_index("c")
    s = lax.axis_index("s")
    wid = c * 16 + s
    pltpu.sync_copy(pk_hbm.at[pl.ds(wid * CHUNKS_PER_TILE, CHUNKS_PER_TILE)], pk_v)
    pltpu.sync_copy(ones_hbm, ones_v)
    pltpu.sync_copy(z1_hbm, acc.at[pl.ds(s * ROWS_PER_TILE, ROWS_PER_TILE)])
    _unpack_edges(pk_v, src_v, dst_v)
    plsc.subcore_barrier()
    for i in range(CHUNKS_PER_TILE):
        pltpu.sync_copy(ones_v, acc.at[dst_v.at[i]], add=True)
    plsc.subcore_barrier()
    rb = s * ROWS_PER_TILE
    pltpu.sync_copy(acc.at[pl.ds(rb, ROWS_PER_TILE)],
                    deg_hbm.at[pl.ds(c * N_PAD + rb, ROWS_PER_TILE)])


def _agg_pass(g_hbm, out_hbm, z_hbm, src_v, dst_v, bufs, sems, acc, c, rb):
    for k in range(ROWS_PER_TILE // CHUNK):
        pltpu.sync_copy(z_hbm, acc.at[pl.ds(rb + k * CHUNK, CHUNK)])
    plsc.subcore_barrier()
    descs = [None, None]
    descs[0] = pltpu.async_copy(g_hbm.at[src_v.at[0]], bufs[0], sems[0])
    for i in range(CHUNKS_PER_TILE):
        cur = i % 2
        nxt = (i + 1) % 2
        if i + 1 < CHUNKS_PER_TILE:
            descs[nxt] = pltpu.async_copy(
                g_hbm.at[src_v.at[i + 1]], bufs[nxt], sems[nxt])
        descs[cur].wait()
        pltpu.sync_copy(bufs[cur], acc.at[dst_v.at[i]], add=True)
    plsc.subcore_barrier()
    for k in range(ROWS_PER_TILE // CHUNK):
        pltpu.sync_copy(acc.at[pl.ds(rb + k * CHUNK, CHUNK)],
                        out_hbm.at[c, pl.ds(rb + k * CHUNK, CHUNK)])
    plsc.subcore_barrier()


@functools.partial(
    pl.kernel,
    out_type=jax.ShapeDtypeStruct((2, 2, N_PAD, DH), jnp.float32),
    mesh=_mesh,
    compiler_params=pltpu.CompilerParams(use_tc_tiling_on_sc=False),
    scratch_types=[
        pltpu.VMEM((CHUNKS_PER_TILE, CHUNK), jnp.int32),
        pltpu.VMEM((CHUNKS_PER_TILE, CHUNK), jnp.int32),
        pltpu.VMEM((CHUNKS_PER_TILE, CHUNK), jnp.int32),
        pltpu.VMEM((CHUNK, DH), jnp.float32),
        pltpu.VMEM((CHUNK, DH), jnp.float32),
        pltpu.VMEM_SHARED((N_PAD, DH), jnp.float32),
        pltpu.SemaphoreType.DMA,
        pltpu.SemaphoreType.DMA,
    ],
)
def _sc_agg_wide(ga_hbm, gb_hbm, pk_hbm, z_hbm, out_hbm,
                 pk_v, src_v, dst_v, buf_a, buf_b, acc, sem_a, sem_b):
    c = lax.axis_index("c")
    s = lax.axis_index("s")
    wid = c * 16 + s
    pltpu.sync_copy(pk_hbm.at[pl.ds(wid * CHUNKS_PER_TILE, CHUNKS_PER_TILE)], pk_v)
    _unpack_edges(pk_v, src_v, dst_v)
    rb = s * ROWS_PER_TILE
    bufs = (buf_a, buf_b)
    sems = (sem_a, sem_b)
    _agg_pass(ga_hbm, out_hbm.at[0], z_hbm, src_v, dst_v, bufs, sems, acc, c, rb)
    _agg_pass(gb_hbm, out_hbm.at[1], z_hbm, src_v, dst_v, bufs, sems, acc, c, rb)


@functools.partial(
    pl.kernel,
    out_type=jax.ShapeDtypeStruct((2, N_PAD, DO), jnp.float32),
    mesh=_mesh,
    compiler_params=pltpu.CompilerParams(use_tc_tiling_on_sc=False),
    scratch_types=[
        pltpu.VMEM((CHUNKS_PER_TILE, CHUNK), jnp.int32),
        pltpu.VMEM((CHUNKS_PER_TILE, CHUNK), jnp.int32),
        pltpu.VMEM((CHUNKS_PER_TILE, CHUNK), jnp.int32),
        pltpu.VMEM((CHUNK, DO), jnp.float32),
        pltpu.VMEM((CHUNK, DO), jnp.float32),
        pltpu.VMEM_SHARED((N_PAD, DO), jnp.float32),
        pltpu.SemaphoreType.DMA,
        pltpu.SemaphoreType.DMA,
    ],
)
def _sc_agg_narrow(g_hbm, pk_hbm, z_hbm, out_hbm,
                   pk_v, src_v, dst_v, buf_a, buf_b, acc, sem_a, sem_b):
    c = lax.axis_index("c")
    s = lax.axis_index("s")
    wid = c * 16 + s
    pltpu.sync_copy(pk_hbm.at[pl.ds(wid * CHUNKS_PER_TILE, CHUNKS_PER_TILE)], pk_v)
    _unpack_edges(pk_v, src_v, dst_v)
    rb = s * ROWS_PER_TILE
    _agg_pass(g_hbm, out_hbm, z_hbm, src_v, dst_v,
              (buf_a, buf_b), (sem_a, sem_b), acc, c, rb)


_R = 1024
_GRID = N_PAD // _R


def _tc1_body(x_ref, fi_ref, w1_ref, d0_ref, d1_ref, ga_ref, gb_ref, s_ref):
    deg = d0_ref[...] + d1_ref[...] + 1.0
    s = lax.rsqrt(deg)
    xw = x_ref[...] * jax.nn.sigmoid(fi_ref[...])
    h = jnp.dot(xw, w1_ref[...], preferred_element_type=jnp.float32)
    g = h * s
    ga_ref[...] = g[:, :DH]
    gb_ref[...] = g[:, DH:]
    s_ref[...] = s


def _tc1(x_p, fi, w1, d0, d1):
    return pl.pallas_call(
        _tc1_body,
        grid=(_GRID,),
        in_specs=[
            pl.BlockSpec((_R, D), lambda i: (i, 0)),
            pl.BlockSpec((1, D), lambda i: (0, 0)),
            pl.BlockSpec((D, D), lambda i: (0, 0)),
            pl.BlockSpec((_R, 1), lambda i: (i, 0)),
            pl.BlockSpec((_R, 1), lambda i: (i, 0)),
        ],
        out_specs=[
            pl.BlockSpec((_R, DH), lambda i: (i, 0)),
            pl.BlockSpec((_R, DH), lambda i: (i, 0)),
            pl.BlockSpec((_R, 1), lambda i: (i, 0)),
        ],
        out_shape=[
            jax.ShapeDtypeStruct((N_PAD, DH), jnp.float32),
            jax.ShapeDtypeStruct((N_PAD, DH), jnp.float32),
            jax.ShapeDtypeStruct((N_PAD, 1), jnp.float32),
        ],
    )(x_p, fi, w1, d0, d1)


def _tc2_body(a0_ref, a1_ref, b0_ref, b1p_ref, ga_ref, gb_ref, s_ref,
              bias1_ref, w2_ref, g2_ref):
    s = s_ref[...]
    za = s * (a0_ref[...] + a1_ref[...] + ga_ref[...]) + bias1_ref[:, :DH]
    zb = s * (b0_ref[...] + b1p_ref[...] + gb_ref[...]) + bias1_ref[:, DH:]
    ha = jnp.maximum(za, 0.0)
    hb = jnp.maximum(zb, 0.0)
    p = (jnp.dot(ha, w2_ref[:DH, :], preferred_element_type=jnp.float32)
         + jnp.dot(hb, w2_ref[DH:, :], preferred_element_type=jnp.float32))
    g2_ref[...] = p * s


def _tc2(a0, a1, b0, b1p, ga, gb, s, bias1, w2p):
    half = pl.BlockSpec((_R, DH), lambda i: (i, 0))
    return pl.pallas_call(
        _tc2_body,
        grid=(_GRID,),
        in_specs=[
            half, half, half, half, half, half,
            pl.BlockSpec((_R, 1), lambda i: (i, 0)),
            pl.BlockSpec((1, D), lambda i: (0, 0)),
            pl.BlockSpec((D, DO), lambda i: (0, 0)),
        ],
        out_specs=pl.BlockSpec((_R, DO), lambda i: (i, 0)),
        out_shape=jax.ShapeDtypeStruct((N_PAD, DO), jnp.float32),
    )(a0, a1, b0, b1p, ga, gb, s, bias1, w2p)


def _tc3_body(q0_ref, q1_ref, g2_ref, s_ref, b2_ref, o_ref):
    o_ref[...] = s_ref[...] * (q0_ref[...] + q1_ref[...] + g2_ref[...]) + b2_ref[...]


def _tc3(q0, q1, g2, s, b2p):
    return pl.pallas_call(
        _tc3_body,
        grid=(_GRID,),
        in_specs=[
            pl.BlockSpec((_R, DO), lambda i: (i, 0)),
            pl.BlockSpec((_R, DO), lambda i: (i, 0)),
            pl.BlockSpec((_R, DO), lambda i: (i, 0)),
            pl.BlockSpec((_R, 1), lambda i: (i, 0)),
            pl.BlockSpec((1, DO), lambda i: (0, 0)),
        ],
        out_specs=pl.BlockSpec((_R, DO), lambda i: (i, 0)),
        out_shape=jax.ShapeDtypeStruct((N_PAD, DO), jnp.float32),
    )(q0, q1, g2, s, b2p)


def kernel(x, edge_index, feature_importance, W1, b1, W2, b2):
    f32 = jnp.float32
    pad_e = E_PAD - E
    src = jnp.concatenate([edge_index[0], jnp.full((pad_e,), N, jnp.int32)])
    dst = jnp.concatenate([edge_index[1], jnp.full((pad_e,), N, jnp.int32)])
    packed = ((dst << SHIFT) | src).reshape(ROW_BLOCKS, CHUNK)

    x_p = jnp.zeros((N_PAD, D), f32).at[:N].set(x)
    fi = feature_importance.reshape(1, D)
    b1r = b1.reshape(1, D)
    w2p = jnp.zeros((D, DO), f32).at[:, : W2.shape[1]].set(W2)
    b2p = jnp.zeros((1, DO), f32).at[0, : b2.shape[0]].set(b2)

    ones_c = jnp.ones((CHUNK,), f32)
    z1 = jnp.zeros((ROWS_PER_TILE,), f32)
    z_half = jnp.zeros((CHUNK, DH), f32)
    z_narrow = jnp.zeros((CHUNK, DO), f32)

    deg_parts = _sc_degree(packed, ones_c, z1).reshape(2, N_PAD)
    d0 = deg_parts[0].reshape(N_PAD, 1)
    d1 = deg_parts[1].reshape(N_PAD, 1)

    ga, gb, s = _tc1(x_p, fi, W1, d0, d1)

    agg1 = _sc_agg_wide(ga, gb, packed, z_half)

    g2 = _tc2(agg1[0, 0], agg1[0, 1], agg1[1, 0], agg1[1, 1],
              ga, gb, s, b1r, w2p)

    agg2 = _sc_agg_narrow(g2, packed, z_narrow)

    out = _tc3(agg2[0], agg2[1], g2, s, b2p)
    return out[:N, : W2.shape[1]]


# speedup vs baseline: 17.9358x; 1.3999x over previous
import functools

import jax
import jax.numpy as jnp
from jax import lax
from jax.experimental import pallas as pl
from jax.experimental.pallas import tpu as pltpu
from jax.experimental.pallas import tpu_sc as plsc

N = 10000
E = 320000
D = 128
DH = 64
DO = 8

N_PAD = 10240
ROWS_PER_TILE = N_PAD // 16
CHUNK = 128
CHUNKS_PER_TILE = 80
EPT = CHUNKS_PER_TILE * CHUNK
E_PAD = EPT * 32
ROW_BLOCKS = E_PAD // CHUNK

SHIFT = 14
MASK = (1 << SHIFT) - 1

_mesh = plsc.VectorSubcoreMesh(core_axis_name="c", subcore_axis_name="s")


def _unpack_edges(packed_v, src_v, dst_v, n_chunks):
    def body(i, _):
        for k in range(CHUNK // 16):
            v = packed_v[i, pl.ds(k * 16, 16)]
            src_v[i, pl.ds(k * 16, 16)] = v & MASK
            dst_v[i, pl.ds(k * 16, 16)] = lax.shift_right_logical(v, SHIFT)
        return 0

    lax.fori_loop(0, n_chunks, body, 0)


@functools.partial(
    pl.kernel,
    out_type=jax.ShapeDtypeStruct((2 * N_PAD,), jnp.float32),
    mesh=_mesh,
    compiler_params=pltpu.CompilerParams(use_tc_tiling_on_sc=False),
    scratch_types=[
        pltpu.VMEM((CHUNKS_PER_TILE, CHUNK), jnp.int32),
        pltpu.VMEM((CHUNKS_PER_TILE, CHUNK), jnp.int32),
        pltpu.VMEM((CHUNKS_PER_TILE, CHUNK), jnp.int32),
        pltpu.VMEM((CHUNK,), jnp.float32),
        pltpu.VMEM_SHARED((N_PAD,), jnp.float32),
    ],
)
def _sc_degree(pk_hbm, ones_hbm, z1_hbm, deg_hbm, pk_v, src_v, dst_v, ones_v, acc):
    c = lax.axis_index("c")
    s = lax.axis_index("s")
    wid = c * 16 + s
    pltpu.sync_copy(pk_hbm.at[pl.ds(wid * CHUNKS_PER_TILE, CHUNKS_PER_TILE)], pk_v)
    pltpu.sync_copy(ones_hbm, ones_v)
    pltpu.sync_copy(z1_hbm, acc.at[pl.ds(s * ROWS_PER_TILE, ROWS_PER_TILE)])
    _unpack_edges(pk_v, src_v, dst_v, CHUNKS_PER_TILE)
    plsc.subcore_barrier()
    for i in range(CHUNKS_PER_TILE):
        pltpu.sync_copy(ones_v, acc.at[dst_v.at[i]], add=True)
    plsc.subcore_barrier()
    rb = s * ROWS_PER_TILE
    pltpu.sync_copy(acc.at[pl.ds(rb, ROWS_PER_TILE)],
                    deg_hbm.at[pl.ds(c * N_PAD + rb, ROWS_PER_TILE)])


def _agg_pass(g_hbm, out_hbm, z_hbm, src_v, dst_v, bufs, sems, acc, c, rb):
    for k in range(ROWS_PER_TILE // CHUNK):
        pltpu.sync_copy(z_hbm, acc.at[pl.ds(rb + k * CHUNK, CHUNK)])
    plsc.subcore_barrier()
    descs = [None, None]
    descs[0] = pltpu.async_copy(g_hbm.at[src_v.at[0]], bufs[0], sems[0])
    for i in range(CHUNKS_PER_TILE):
        cur = i % 2
        nxt = (i + 1) % 2
        if i + 1 < CHUNKS_PER_TILE:
            descs[nxt] = pltpu.async_copy(
                g_hbm.at[src_v.at[i + 1]], bufs[nxt], sems[nxt])
        descs[cur].wait()
        pltpu.sync_copy(bufs[cur], acc.at[dst_v.at[i]], add=True)
    plsc.subcore_barrier()
    for k in range(ROWS_PER_TILE // CHUNK):
        pltpu.sync_copy(acc.at[pl.ds(rb + k * CHUNK, CHUNK)],
                        out_hbm.at[c, pl.ds(rb + k * CHUNK, CHUNK)])
    plsc.subcore_barrier()


TILE_CHUNKS = ROW_BLOCKS // 16


@functools.partial(
    pl.kernel,
    out_type=jax.ShapeDtypeStruct((2, N_PAD, DH), jnp.float32),
    mesh=_mesh,
    compiler_params=pltpu.CompilerParams(use_tc_tiling_on_sc=False),
    scratch_types=[
        pltpu.VMEM((TILE_CHUNKS, CHUNK), jnp.int32),
        pltpu.VMEM((TILE_CHUNKS, CHUNK), jnp.int32),
        pltpu.VMEM((TILE_CHUNKS, CHUNK), jnp.int32),
        pltpu.VMEM((CHUNK, DH), jnp.float32),
        pltpu.VMEM((CHUNK, DH), jnp.float32),
        pltpu.VMEM_SHARED((N_PAD, DH), jnp.float32),
        pltpu.SemaphoreType.DMA,
        pltpu.SemaphoreType.DMA,
    ],
)
def _sc_agg_wide(g_hbm, pk_hbm, z_hbm, out_hbm,
                 pk_v, src_v, dst_v, buf_a, buf_b, acc, sem_a, sem_b):
    c = lax.axis_index("c")
    s = lax.axis_index("s")
    pltpu.sync_copy(pk_hbm.at[pl.ds(s * TILE_CHUNKS, TILE_CHUNKS)], pk_v)
    _unpack_edges(pk_v, src_v, dst_v, TILE_CHUNKS)
    rb = s * ROWS_PER_TILE
    for k in range(ROWS_PER_TILE // CHUNK):
        pltpu.sync_copy(z_hbm, acc.at[pl.ds(rb + k * CHUNK, CHUNK)])
    plsc.subcore_barrier()
    gsrc = g_hbm.at[c]
    bufs = (buf_a, buf_b)
    sems = (sem_a, sem_b)
    descs = [None, None]
    descs[0] = pltpu.async_copy(gsrc.at[src_v.at[0]], bufs[0], sems[0])
    for i in range(TILE_CHUNKS):
        cur = i % 2
        nxt = (i + 1) % 2
        if i + 1 < TILE_CHUNKS:
            descs[nxt] = pltpu.async_copy(
                gsrc.at[src_v.at[i + 1]], bufs[nxt], sems[nxt])
        descs[cur].wait()
        pltpu.sync_copy(bufs[cur], acc.at[dst_v.at[i]], add=True)
    plsc.subcore_barrier()
    for k in range(ROWS_PER_TILE // CHUNK):
        pltpu.sync_copy(acc.at[pl.ds(rb + k * CHUNK, CHUNK)],
                        out_hbm.at[c, pl.ds(rb + k * CHUNK, CHUNK)])


@functools.partial(
    pl.kernel,
    out_type=jax.ShapeDtypeStruct((2, N_PAD, DO), jnp.float32),
    mesh=_mesh,
    compiler_params=pltpu.CompilerParams(use_tc_tiling_on_sc=False),
    scratch_types=[
        pltpu.VMEM((CHUNKS_PER_TILE, CHUNK), jnp.int32),
        pltpu.VMEM((CHUNKS_PER_TILE, CHUNK), jnp.int32),
        pltpu.VMEM((CHUNKS_PER_TILE, CHUNK), jnp.int32),
        pltpu.VMEM((CHUNK, DO), jnp.float32),
        pltpu.VMEM((CHUNK, DO), jnp.float32),
        pltpu.VMEM_SHARED((N_PAD, DO), jnp.float32),
        pltpu.SemaphoreType.DMA,
        pltpu.SemaphoreType.DMA,
    ],
)
def _sc_agg_narrow(g_hbm, pk_hbm, z_hbm, out_hbm,
                   pk_v, src_v, dst_v, buf_a, buf_b, acc, sem_a, sem_b):
    c = lax.axis_index("c")
    s = lax.axis_index("s")
    wid = c * 16 + s
    pltpu.sync_copy(pk_hbm.at[pl.ds(wid * CHUNKS_PER_TILE, CHUNKS_PER_TILE)], pk_v)
    _unpack_edges(pk_v, src_v, dst_v, CHUNKS_PER_TILE)
    rb = s * ROWS_PER_TILE
    _agg_pass(g_hbm, out_hbm, z_hbm, src_v, dst_v,
              (buf_a, buf_b), (sem_a, sem_b), acc, c, rb)


_R = 1024
_GRID = N_PAD // _R


def _tc1_body(x_ref, fi_ref, w1_ref, d0_ref, d1_ref, ga_ref, gb_ref, s_ref):
    deg = d0_ref[...] + d1_ref[...] + 1.0
    s = lax.rsqrt(deg)
    xw = x_ref[...] * jax.nn.sigmoid(fi_ref[...])
    h = jnp.dot(xw, w1_ref[...], preferred_element_type=jnp.float32)
    g = h * s
    ga_ref[...] = g[:, :DH]
    gb_ref[...] = g[:, DH:]
    s_ref[...] = s


def _tc1(x_p, fi, w1, d0, d1):
    return pl.pallas_call(
        _tc1_body,
        grid=(_GRID,),
        in_specs=[
            pl.BlockSpec((_R, D), lambda i: (i, 0)),
            pl.BlockSpec((1, D), lambda i: (0, 0)),
            pl.BlockSpec((D, D), lambda i: (0, 0)),
            pl.BlockSpec((_R, 1), lambda i: (i, 0)),
            pl.BlockSpec((_R, 1), lambda i: (i, 0)),
        ],
        out_specs=[
            pl.BlockSpec((_R, DH), lambda i: (i, 0)),
            pl.BlockSpec((_R, DH), lambda i: (i, 0)),
            pl.BlockSpec((_R, 1), lambda i: (i, 0)),
        ],
        out_shape=[
            jax.ShapeDtypeStruct((N_PAD, DH), jnp.float32),
            jax.ShapeDtypeStruct((N_PAD, DH), jnp.float32),
            jax.ShapeDtypeStruct((N_PAD, 1), jnp.float32),
        ],
    )(x_p, fi, w1, d0, d1)


def _tc2_body(aa_ref, ab_ref, ga_ref, gb_ref, s_ref,
              bias1_ref, w2_ref, g2_ref):
    s = s_ref[...]
    za = s * (aa_ref[...] + ga_ref[...]) + bias1_ref[:, :DH]
    zb = s * (ab_ref[...] + gb_ref[...]) + bias1_ref[:, DH:]
    ha = jnp.maximum(za, 0.0)
    hb = jnp.maximum(zb, 0.0)
    p = (jnp.dot(ha, w2_ref[:DH, :], preferred_element_type=jnp.float32)
         + jnp.dot(hb, w2_ref[DH:, :], preferred_element_type=jnp.float32))
    g2_ref[...] = p * s


def _tc2(aa, ab, ga, gb, s, bias1, w2p):
    half = pl.BlockSpec((_R, DH), lambda i: (i, 0))
    return pl.pallas_call(
        _tc2_body,
        grid=(_GRID,),
        in_specs=[
            half, half, half, half,
            pl.BlockSpec((_R, 1), lambda i: (i, 0)),
            pl.BlockSpec((1, D), lambda i: (0, 0)),
            pl.BlockSpec((D, DO), lambda i: (0, 0)),
        ],
        out_specs=pl.BlockSpec((_R, DO), lambda i: (i, 0)),
        out_shape=jax.ShapeDtypeStruct((N_PAD, DO), jnp.float32),
    )(aa, ab, ga, gb, s, bias1, w2p)


def _tc3_body(q0_ref, q1_ref, g2_ref, s_ref, b2_ref, o_ref):
    o_ref[...] = s_ref[...] * (q0_ref[...] + q1_ref[...] + g2_ref[...]) + b2_ref[...]


def _tc3(q0, q1, g2, s, b2p):
    return pl.pallas_call(
        _tc3_body,
        grid=(_GRID,),
        in_specs=[
            pl.BlockSpec((_R, DO), lambda i: (i, 0)),
            pl.BlockSpec((_R, DO), lambda i: (i, 0)),
            pl.BlockSpec((_R, DO), lambda i: (i, 0)),
            pl.BlockSpec((_R, 1), lambda i: (i, 0)),
            pl.BlockSpec((1, DO), lambda i: (0, 0)),
        ],
        out_specs=pl.BlockSpec((_R, DO), lambda i: (i, 0)),
        out_shape=jax.ShapeDtypeStruct((N_PAD, DO), jnp.float32),
    )(q0, q1, g2, s, b2p)


def kernel(x, edge_index, feature_importance, W1, b1, W2, b2):
    f32 = jnp.float32
    pad_e = E_PAD - E
    src = jnp.concatenate([edge_index[0], jnp.full((pad_e,), N, jnp.int32)])
    dst = jnp.concatenate([edge_index[1], jnp.full((pad_e,), N, jnp.int32)])
    packed = ((dst << SHIFT) | src).reshape(ROW_BLOCKS, CHUNK)

    x_p = jnp.zeros((N_PAD, D), f32).at[:N].set(x)
    fi = feature_importance.reshape(1, D)
    b1r = b1.reshape(1, D)
    w2p = jnp.zeros((D, DO), f32).at[:, : W2.shape[1]].set(W2)
    b2p = jnp.zeros((1, DO), f32).at[0, : b2.shape[0]].set(b2)

    ones_c = jnp.ones((CHUNK,), f32)
    z1 = jnp.zeros((ROWS_PER_TILE,), f32)
    z_half = jnp.zeros((CHUNK, DH), f32)
    z_narrow = jnp.zeros((CHUNK, DO), f32)

    deg_parts = _sc_degree(packed, ones_c, z1).reshape(2, N_PAD)
    d0 = deg_parts[0].reshape(N_PAD, 1)
    d1 = deg_parts[1].reshape(N_PAD, 1)

    ga, gb, s = _tc1(x_p, fi, W1, d0, d1)
    g_stack = jnp.stack([ga, gb])

    agg1 = _sc_agg_wide(g_stack, packed, z_half)

    g2 = _tc2(agg1[0], agg1[1], ga, gb, s, b1r, w2p)

    agg2 = _sc_agg_narrow(g2, packed, z_narrow)

    out = _tc3(agg2[0], agg2[1], g2, s, b2p)
    return out[:N, : W2.shape[1]]


# speedup vs baseline: 19.6008x; 1.0928x over previous
import functools

import jax
import jax.numpy as jnp
from jax import lax
from jax.experimental import pallas as pl
from jax.experimental.pallas import tpu as pltpu
from jax.experimental.pallas import tpu_sc as plsc

N = 10000
E = 320000
D = 128
DH = 64
DO = 8

N_PAD = 10240
ROWS_PER_TILE = N_PAD // 16
CHUNK = 128
CHUNKS_PER_TILE = 80
EPT = CHUNKS_PER_TILE * CHUNK
E_PAD = EPT * 32
ROW_BLOCKS = E_PAD // CHUNK

SHIFT = 14
MASK = (1 << SHIFT) - 1

_mesh = plsc.VectorSubcoreMesh(core_axis_name="c", subcore_axis_name="s")


def _unpack_edges(packed_v, src_v, dst_v, n_chunks):
    def body(i, _):
        for k in range(CHUNK // 16):
            v = packed_v[i, pl.ds(k * 16, 16)]
            src_v[i, pl.ds(k * 16, 16)] = v & MASK
            dst_v[i, pl.ds(k * 16, 16)] = lax.shift_right_logical(v, SHIFT)
        return 0

    lax.fori_loop(0, n_chunks, body, 0)


@functools.partial(
    pl.kernel,
    out_type=jax.ShapeDtypeStruct((2 * N_PAD,), jnp.float32),
    mesh=_mesh,
    compiler_params=pltpu.CompilerParams(use_tc_tiling_on_sc=False),
    scratch_types=[
        pltpu.VMEM((CHUNKS_PER_TILE, CHUNK), jnp.int32),
        pltpu.VMEM((CHUNKS_PER_TILE, CHUNK), jnp.int32),
        pltpu.VMEM((CHUNKS_PER_TILE, CHUNK), jnp.int32),
        pltpu.VMEM((CHUNK,), jnp.float32),
        pltpu.VMEM_SHARED((N_PAD,), jnp.float32),
    ],
)
def _sc_degree(pk_hbm, ones_hbm, z1_hbm, deg_hbm, pk_v, src_v, dst_v, ones_v, acc):
    c = lax.axis_index("c")
    s = lax.axis_index("s")
    wid = c * 16 + s
    pltpu.sync_copy(pk_hbm.at[pl.ds(wid * CHUNKS_PER_TILE, CHUNKS_PER_TILE)], pk_v)
    pltpu.sync_copy(ones_hbm, ones_v)
    pltpu.sync_copy(z1_hbm, acc.at[pl.ds(s * ROWS_PER_TILE, ROWS_PER_TILE)])
    _unpack_edges(pk_v, src_v, dst_v, CHUNKS_PER_TILE)
    plsc.subcore_barrier()
    for i in range(CHUNKS_PER_TILE):
        pltpu.sync_copy(ones_v, acc.at[dst_v.at[i]], add=True)
    plsc.subcore_barrier()
    rb = s * ROWS_PER_TILE
    pltpu.sync_copy(acc.at[pl.ds(rb, ROWS_PER_TILE)],
                    deg_hbm.at[pl.ds(c * N_PAD + rb, ROWS_PER_TILE)])


def _agg_pass(g_hbm, out_hbm, z_hbm, src_v, dst_v, bufs, sems, acc, c, rb):
    for k in range(ROWS_PER_TILE // CHUNK):
        pltpu.sync_copy(z_hbm, acc.at[pl.ds(rb + k * CHUNK, CHUNK)])
    plsc.subcore_barrier()
    descs = [None, None]
    descs[0] = pltpu.async_copy(g_hbm.at[src_v.at[0]], bufs[0], sems[0])
    for i in range(CHUNKS_PER_TILE):
        cur = i % 2
        nxt = (i + 1) % 2
        if i + 1 < CHUNKS_PER_TILE:
            descs[nxt] = pltpu.async_copy(
                g_hbm.at[src_v.at[i + 1]], bufs[nxt], sems[nxt])
        descs[cur].wait()
        pltpu.sync_copy(bufs[cur], acc.at[dst_v.at[i]], add=True)
    plsc.subcore_barrier()
    for k in range(ROWS_PER_TILE // CHUNK):
        pltpu.sync_copy(acc.at[pl.ds(rb + k * CHUNK, CHUNK)],
                        out_hbm.at[c, pl.ds(rb + k * CHUNK, CHUNK)])
    plsc.subcore_barrier()


TILE_CHUNKS = ROW_BLOCKS // 16


DW = DH // 2
HI_MASK = -65536


def _expand_chunk(gbuf, fbuf):
    def body(r4, _):
        for u in range(4):
            r = r4 * 4 + u
            for k in range(2):
                w = gbuf[r, pl.ds(k * 16, 16)]
                lo = plsc.bitcast(lax.shift_left(w, 16), jnp.float32)
                hi = plsc.bitcast(w & HI_MASK, jnp.float32)
                fbuf[r, pl.ds(k * 16, 16)] = lo
                fbuf[r, pl.ds(32 + k * 16, 16)] = hi
        return 0

    lax.fori_loop(0, CHUNK // 4, body, 0)


@functools.partial(
    pl.kernel,
    out_type=jax.ShapeDtypeStruct((2, N_PAD, DH), jnp.float32),
    mesh=_mesh,
    compiler_params=pltpu.CompilerParams(
        use_tc_tiling_on_sc=False, needs_layout_passes=False),
    scratch_types=[
        pltpu.VMEM((TILE_CHUNKS, CHUNK), jnp.int32),
        pltpu.VMEM((TILE_CHUNKS, CHUNK), jnp.int32),
        pltpu.VMEM((TILE_CHUNKS, CHUNK), jnp.int32),
        pltpu.VMEM((CHUNK, DW), jnp.int32),
        pltpu.VMEM((CHUNK, DW), jnp.int32),
        pltpu.VMEM((CHUNK, DH), jnp.float32),
        pltpu.VMEM((CHUNK, DH), jnp.float32),
        pltpu.VMEM_SHARED((N_PAD, DH), jnp.float32),
        pltpu.SemaphoreType.DMA,
        pltpu.SemaphoreType.DMA,
        pltpu.SemaphoreType.DMA,
        pltpu.SemaphoreType.DMA,
    ],
)
def _sc_agg_wide(g_hbm, pk_hbm, z_hbm, out_hbm,
                 pk_v, src_v, dst_v, gb_a, gb_b, fb_a, fb_b, acc,
                 gsem_a, gsem_b, ssem_a, ssem_b):
    c = lax.axis_index("c")
    s = lax.axis_index("s")
    pltpu.sync_copy(pk_hbm.at[pl.ds(s * TILE_CHUNKS, TILE_CHUNKS)], pk_v)
    _unpack_edges(pk_v, src_v, dst_v, TILE_CHUNKS)
    rb = s * ROWS_PER_TILE
    for k in range(ROWS_PER_TILE // CHUNK):
        pltpu.sync_copy(z_hbm, acc.at[pl.ds(rb + k * CHUNK, CHUNK)])
    plsc.subcore_barrier()
    gsrc = g_hbm.at[c]
    pltpu.async_copy(gsrc.at[src_v.at[0]], gb_a, gsem_a)
    pltpu.async_copy(gsrc.at[src_v.at[1]], gb_b, gsem_b)

    def chunk_pair(t, _):
        for i, gb, fb, gsem, ssem in (
            (2 * t, gb_a, fb_a, gsem_a, ssem_a),
            (2 * t + 1, gb_b, fb_b, gsem_b, ssem_b),
        ):
            pltpu.make_async_copy(gsrc.at[pl.ds(0, CHUNK)], gb, gsem).wait()
            @pl.when(t > 0)
            def _():
                pltpu.make_async_copy(z_hbm, fb, ssem).wait()

            _expand_chunk(gb, fb)

            @pl.when(t < TILE_CHUNKS // 2 - 1)
            def _():
                pltpu.async_copy(gsrc.at[src_v.at[i + 2]], gb, gsem)

            pltpu.async_copy(fb, acc.at[dst_v.at[i]], ssem, add=True)
        return 0

    lax.fori_loop(0, TILE_CHUNKS // 2, chunk_pair, 0)
    pltpu.make_async_copy(z_hbm, fb_a, ssem_a).wait()
    pltpu.make_async_copy(z_hbm, fb_b, ssem_b).wait()
    plsc.subcore_barrier()
    for k in range(ROWS_PER_TILE // CHUNK):
        pltpu.sync_copy(acc.at[pl.ds(rb + k * CHUNK, CHUNK)],
                        out_hbm.at[c, pl.ds(rb + k * CHUNK, CHUNK)])


@functools.partial(
    pl.kernel,
    out_type=jax.ShapeDtypeStruct((2, N_PAD, DO), jnp.float32),
    mesh=_mesh,
    compiler_params=pltpu.CompilerParams(use_tc_tiling_on_sc=False),
    scratch_types=[
        pltpu.VMEM((CHUNKS_PER_TILE, CHUNK), jnp.int32),
        pltpu.VMEM((CHUNKS_PER_TILE, CHUNK), jnp.int32),
        pltpu.VMEM((CHUNKS_PER_TILE, CHUNK), jnp.int32),
        pltpu.VMEM((CHUNK, DO), jnp.float32),
        pltpu.VMEM((CHUNK, DO), jnp.float32),
        pltpu.VMEM_SHARED((N_PAD, DO), jnp.float32),
        pltpu.SemaphoreType.DMA,
        pltpu.SemaphoreType.DMA,
    ],
)
def _sc_agg_narrow(g_hbm, pk_hbm, z_hbm, out_hbm,
                   pk_v, src_v, dst_v, buf_a, buf_b, acc, sem_a, sem_b):
    c = lax.axis_index("c")
    s = lax.axis_index("s")
    wid = c * 16 + s
    pltpu.sync_copy(pk_hbm.at[pl.ds(wid * CHUNKS_PER_TILE, CHUNKS_PER_TILE)], pk_v)
    _unpack_edges(pk_v, src_v, dst_v, CHUNKS_PER_TILE)
    rb = s * ROWS_PER_TILE
    _agg_pass(g_hbm, out_hbm, z_hbm, src_v, dst_v,
              (buf_a, buf_b), (sem_a, sem_b), acc, c, rb)


_R = 1024
_GRID = N_PAD // _R


def _pack_bf16_pairs(x):
    u = lax.bitcast_convert_type(x, jnp.int32)
    r = u + 0x7FFF + (lax.shift_right_logical(u, 16) & 1)
    r_lo = r[:, :DW]
    r_hi = r[:, DW:]
    return lax.shift_right_logical(r_lo, 16) | (r_hi & HI_MASK)


def _unpack_bf16_pairs(w):
    lo = lax.bitcast_convert_type(lax.shift_left(w, 16), jnp.float32)
    hi = lax.bitcast_convert_type(w & HI_MASK, jnp.float32)
    return jnp.concatenate([lo, hi], axis=1)


def _tc1_body(x_ref, fi_ref, w1_ref, d0_ref, d1_ref, pa_ref, pb_ref, s_ref):
    deg = d0_ref[...] + d1_ref[...] + 1.0
    s = lax.rsqrt(deg)
    xw = x_ref[...] * jax.nn.sigmoid(fi_ref[...])
    h = jnp.dot(xw, w1_ref[...], preferred_element_type=jnp.float32)
    g = h * s
    pa_ref[...] = _pack_bf16_pairs(g[:, :DH])
    pb_ref[...] = _pack_bf16_pairs(g[:, DH:])
    s_ref[...] = s


def _tc1(x_p, fi, w1, d0, d1):
    return pl.pallas_call(
        _tc1_body,
        grid=(_GRID,),
        in_specs=[
            pl.BlockSpec((_R, D), lambda i: (i, 0)),
            pl.BlockSpec((1, D), lambda i: (0, 0)),
            pl.BlockSpec((D, D), lambda i: (0, 0)),
            pl.BlockSpec((_R, 1), lambda i: (i, 0)),
            pl.BlockSpec((_R, 1), lambda i: (i, 0)),
        ],
        out_specs=[
            pl.BlockSpec((_R, DW), lambda i: (i, 0)),
            pl.BlockSpec((_R, DW), lambda i: (i, 0)),
            pl.BlockSpec((_R, 1), lambda i: (i, 0)),
        ],
        out_shape=[
            jax.ShapeDtypeStruct((N_PAD, DW), jnp.int32),
            jax.ShapeDtypeStruct((N_PAD, DW), jnp.int32),
            jax.ShapeDtypeStruct((N_PAD, 1), jnp.float32),
        ],
    )(x_p, fi, w1, d0, d1)


def _tc2_body(aa_ref, ab_ref, pa_ref, pb_ref, s_ref,
              bias1_ref, w2_ref, g2_ref):
    s = s_ref[...]
    ga = _unpack_bf16_pairs(pa_ref[...])
    gb = _unpack_bf16_pairs(pb_ref[...])
    za = s * (aa_ref[...] + ga) + bias1_ref[:, :DH]
    zb = s * (ab_ref[...] + gb) + bias1_ref[:, DH:]
    ha = jnp.maximum(za, 0.0)
    hb = jnp.maximum(zb, 0.0)
    p = (jnp.dot(ha, w2_ref[:DH, :], preferred_element_type=jnp.float32)
         + jnp.dot(hb, w2_ref[DH:, :], preferred_element_type=jnp.float32))
    g2_ref[...] = p * s


def _tc2(aa, ab, pa, pb, s, bias1, w2p):
    half = pl.BlockSpec((_R, DH), lambda i: (i, 0))
    packed = pl.BlockSpec((_R, DW), lambda i: (i, 0))
    return pl.pallas_call(
        _tc2_body,
        grid=(_GRID,),
        in_specs=[
            half, half, packed, packed,
            pl.BlockSpec((_R, 1), lambda i: (i, 0)),
            pl.BlockSpec((1, D), lambda i: (0, 0)),
            pl.BlockSpec((D, DO), lambda i: (0, 0)),
        ],
        out_specs=pl.BlockSpec((_R, DO), lambda i: (i, 0)),
        out_shape=jax.ShapeDtypeStruct((N_PAD, DO), jnp.float32),
    )(aa, ab, pa, pb, s, bias1, w2p)


def _tc3_body(q0_ref, q1_ref, g2_ref, s_ref, b2_ref, o_ref):
    o_ref[...] = s_ref[...] * (q0_ref[...] + q1_ref[...] + g2_ref[...]) + b2_ref[...]


def _tc3(q0, q1, g2, s, b2p):
    return pl.pallas_call(
        _tc3_body,
        grid=(_GRID,),
        in_specs=[
            pl.BlockSpec((_R, DO), lambda i: (i, 0)),
            pl.BlockSpec((_R, DO), lambda i: (i, 0)),
            pl.BlockSpec((_R, DO), lambda i: (i, 0)),
            pl.BlockSpec((_R, 1), lambda i: (i, 0)),
            pl.BlockSpec((1, DO), lambda i: (0, 0)),
        ],
        out_specs=pl.BlockSpec((_R, DO), lambda i: (i, 0)),
        out_shape=jax.ShapeDtypeStruct((N_PAD, DO), jnp.float32),
    )(q0, q1, g2, s, b2p)


def kernel(x, edge_index, feature_importance, W1, b1, W2, b2):
    f32 = jnp.float32
    pad_e = E_PAD - E
    src = jnp.concatenate([edge_index[0], jnp.full((pad_e,), N, jnp.int32)])
    dst = jnp.concatenate([edge_index[1], jnp.full((pad_e,), N, jnp.int32)])
    packed = ((dst << SHIFT) | src).reshape(ROW_BLOCKS, CHUNK)

    x_p = jnp.zeros((N_PAD, D), f32).at[:N].set(x)
    fi = feature_importance.reshape(1, D)
    b1r = b1.reshape(1, D)
    w2p = jnp.zeros((D, DO), f32).at[:, : W2.shape[1]].set(W2)
    b2p = jnp.zeros((1, DO), f32).at[0, : b2.shape[0]].set(b2)

    ones_c = jnp.ones((CHUNK,), f32)
    z1 = jnp.zeros((ROWS_PER_TILE,), f32)
    z_half = jnp.zeros((CHUNK, DH), f32)
    z_narrow = jnp.zeros((CHUNK, DO), f32)

    deg_parts = _sc_degree(packed, ones_c, z1).reshape(2, N_PAD)
    d0 = deg_parts[0].reshape(N_PAD, 1)
    d1 = deg_parts[1].reshape(N_PAD, 1)

    pa, pb, s = _tc1(x_p, fi, W1, d0, d1)
    g_stack = jnp.stack([pa, pb])

    agg1 = _sc_agg_wide(g_stack, packed, z_half)

    g2 = _tc2(agg1[0], agg1[1], pa, pb, s, b1r, w2p)

    agg2 = _sc_agg_narrow(g2, packed, z_narrow)

    out = _tc3(agg2[0], agg2[1], g2, s, b2p)
    return out[:N, : W2.shape[1]]


# speedup vs baseline: 19.7043x; 1.0053x over previous
import functools

import jax
import jax.numpy as jnp
from jax import lax
from jax.experimental import pallas as pl
from jax.experimental.pallas import tpu as pltpu
from jax.experimental.pallas import tpu_sc as plsc

N = 10000
E = 320000
D = 128
DH = 64
DO = 8

N_PAD = 10240
ROWS_PER_TILE = N_PAD // 16
CHUNK = 128
CHUNKS_PER_TILE = 80
EPT = CHUNKS_PER_TILE * CHUNK
E_PAD = EPT * 32
ROW_BLOCKS = E_PAD // CHUNK

SHIFT = 14
MASK = (1 << SHIFT) - 1

_mesh = plsc.VectorSubcoreMesh(core_axis_name="c", subcore_axis_name="s")


def _unpack_edges(packed_v, src_v, dst_v, n_chunks):
    def body(i, _):
        for k in range(CHUNK // 16):
            v = packed_v[i, pl.ds(k * 16, 16)]
            src_v[i, pl.ds(k * 16, 16)] = v & MASK
            dst_v[i, pl.ds(k * 16, 16)] = lax.shift_right_logical(v, SHIFT)
        return 0

    lax.fori_loop(0, n_chunks, body, 0)


@functools.partial(
    pl.kernel,
    out_type=jax.ShapeDtypeStruct((2 * N_PAD,), jnp.float32),
    mesh=_mesh,
    compiler_params=pltpu.CompilerParams(use_tc_tiling_on_sc=False),
    scratch_types=[
        pltpu.VMEM((CHUNKS_PER_TILE, CHUNK), jnp.int32),
        pltpu.VMEM((CHUNKS_PER_TILE, CHUNK), jnp.int32),
        pltpu.VMEM((CHUNKS_PER_TILE, CHUNK), jnp.int32),
        pltpu.VMEM((CHUNK,), jnp.float32),
        pltpu.VMEM_SHARED((N_PAD,), jnp.float32),
    ],
)
def _sc_degree(pk_hbm, ones_hbm, z1_hbm, deg_hbm, pk_v, src_v, dst_v, ones_v, acc):
    c = lax.axis_index("c")
    s = lax.axis_index("s")
    wid = c * 16 + s
    pltpu.sync_copy(pk_hbm.at[pl.ds(wid * CHUNKS_PER_TILE, CHUNKS_PER_TILE)], pk_v)
    pltpu.sync_copy(ones_hbm, ones_v)
    pltpu.sync_copy(z1_hbm, acc.at[pl.ds(s * ROWS_PER_TILE, ROWS_PER_TILE)])
    _unpack_edges(pk_v, src_v, dst_v, CHUNKS_PER_TILE)
    plsc.subcore_barrier()
    for i in range(CHUNKS_PER_TILE):
        pltpu.sync_copy(ones_v, acc.at[dst_v.at[i]], add=True)
    plsc.subcore_barrier()
    rb = s * ROWS_PER_TILE
    pltpu.sync_copy(acc.at[pl.ds(rb, ROWS_PER_TILE)],
                    deg_hbm.at[pl.ds(c * N_PAD + rb, ROWS_PER_TILE)])


def _agg_pass(g_hbm, out_hbm, z_hbm, src_v, dst_v, bufs, sems, acc, c, rb):
    for k in range(ROWS_PER_TILE // CHUNK):
        pltpu.sync_copy(z_hbm, acc.at[pl.ds(rb + k * CHUNK, CHUNK)])
    plsc.subcore_barrier()
    descs = [None, None]
    descs[0] = pltpu.async_copy(g_hbm.at[src_v.at[0]], bufs[0], sems[0])
    for i in range(CHUNKS_PER_TILE):
        cur = i % 2
        nxt = (i + 1) % 2
        if i + 1 < CHUNKS_PER_TILE:
            descs[nxt] = pltpu.async_copy(
                g_hbm.at[src_v.at[i + 1]], bufs[nxt], sems[nxt])
        descs[cur].wait()
        pltpu.sync_copy(bufs[cur], acc.at[dst_v.at[i]], add=True)
    plsc.subcore_barrier()
    for k in range(ROWS_PER_TILE // CHUNK):
        pltpu.sync_copy(acc.at[pl.ds(rb + k * CHUNK, CHUNK)],
                        out_hbm.at[c, pl.ds(rb + k * CHUNK, CHUNK)])
    plsc.subcore_barrier()


TILE_CHUNKS = ROW_BLOCKS // 16


DW = DH // 2
HI_MASK = -65536


def _expand_chunk(gbuf, fbuf):
    def body(r8, _):
        for u in range(8):
            r = r8 * 8 + u
            for k in range(2):
                w = gbuf[r, pl.ds(k * 16, 16)]
                lo = plsc.bitcast(lax.shift_left(w, 16), jnp.float32)
                hi = plsc.bitcast(w & HI_MASK, jnp.float32)
                fbuf[r, pl.ds(k * 16, 16)] = lo
                fbuf[r, pl.ds(32 + k * 16, 16)] = hi
        return 0

    lax.fori_loop(0, CHUNK // 8, body, 0)


@functools.partial(
    pl.kernel,
    out_type=jax.ShapeDtypeStruct((2, N_PAD, DH), jnp.float32),
    mesh=_mesh,
    compiler_params=pltpu.CompilerParams(
        use_tc_tiling_on_sc=False, needs_layout_passes=False),
    scratch_types=[
        pltpu.VMEM((TILE_CHUNKS, CHUNK), jnp.int32),
        pltpu.VMEM((TILE_CHUNKS, CHUNK), jnp.int32),
        pltpu.VMEM((TILE_CHUNKS, CHUNK), jnp.int32),
        pltpu.VMEM((CHUNK, DW), jnp.int32),
        pltpu.VMEM((CHUNK, DW), jnp.int32),
        pltpu.VMEM((CHUNK, DH), jnp.float32),
        pltpu.VMEM((CHUNK, DH), jnp.float32),
        pltpu.VMEM_SHARED((N_PAD, DH), jnp.float32),
        pltpu.SemaphoreType.DMA,
        pltpu.SemaphoreType.DMA,
        pltpu.SemaphoreType.DMA,
        pltpu.SemaphoreType.DMA,
    ],
)
def _sc_agg_wide(g_hbm, pk_hbm, z_hbm, out_hbm,
                 pk_v, src_v, dst_v, gb_a, gb_b, fb_a, fb_b, acc,
                 gsem_a, gsem_b, ssem_a, ssem_b):
    c = lax.axis_index("c")
    s = lax.axis_index("s")
    pltpu.sync_copy(pk_hbm.at[pl.ds(s * TILE_CHUNKS, TILE_CHUNKS)], pk_v)
    _unpack_edges(pk_v, src_v, dst_v, TILE_CHUNKS)
    rb = s * ROWS_PER_TILE
    for k in range(ROWS_PER_TILE // CHUNK):
        pltpu.sync_copy(z_hbm, acc.at[pl.ds(rb + k * CHUNK, CHUNK)])
    plsc.subcore_barrier()
    gsrc = g_hbm.at[c]
    pltpu.async_copy(gsrc.at[src_v.at[0]], gb_a, gsem_a)
    pltpu.async_copy(gsrc.at[src_v.at[1]], gb_b, gsem_b)

    def chunk_pair(t, _):
        for i, gb, fb, gsem, ssem in (
            (2 * t, gb_a, fb_a, gsem_a, ssem_a),
            (2 * t + 1, gb_b, fb_b, gsem_b, ssem_b),
        ):
            pltpu.make_async_copy(gsrc.at[pl.ds(0, CHUNK)], gb, gsem).wait()
            @pl.when(t > 0)
            def _():
                pltpu.make_async_copy(z_hbm, fb, ssem).wait()

            _expand_chunk(gb, fb)

            @pl.when(t < TILE_CHUNKS // 2 - 1)
            def _():
                pltpu.async_copy(gsrc.at[src_v.at[i + 2]], gb, gsem)

            pltpu.async_copy(fb, acc.at[dst_v.at[i]], ssem, add=True)
        return 0

    lax.fori_loop(0, TILE_CHUNKS // 2, chunk_pair, 0)
    pltpu.make_async_copy(z_hbm, fb_a, ssem_a).wait()
    pltpu.make_async_copy(z_hbm, fb_b, ssem_b).wait()
    plsc.subcore_barrier()
    for k in range(ROWS_PER_TILE // CHUNK):
        pltpu.sync_copy(acc.at[pl.ds(rb + k * CHUNK, CHUNK)],
                        out_hbm.at[c, pl.ds(rb + k * CHUNK, CHUNK)])


@functools.partial(
    pl.kernel,
    out_type=jax.ShapeDtypeStruct((2, N_PAD, DO), jnp.float32),
    mesh=_mesh,
    compiler_params=pltpu.CompilerParams(use_tc_tiling_on_sc=False),
    scratch_types=[
        pltpu.VMEM((CHUNKS_PER_TILE, CHUNK), jnp.int32),
        pltpu.VMEM((CHUNKS_PER_TILE, CHUNK), jnp.int32),
        pltpu.VMEM((CHUNKS_PER_TILE, CHUNK), jnp.int32),
        pltpu.VMEM((CHUNK, DO), jnp.float32),
        pltpu.VMEM((CHUNK, DO), jnp.float32),
        pltpu.VMEM_SHARED((N_PAD, DO), jnp.float32),
        pltpu.SemaphoreType.DMA,
        pltpu.SemaphoreType.DMA,
    ],
)
def _sc_agg_narrow(g_hbm, pk_hbm, z_hbm, out_hbm,
                   pk_v, src_v, dst_v, buf_a, buf_b, acc, sem_a, sem_b):
    c = lax.axis_index("c")
    s = lax.axis_index("s")
    wid = c * 16 + s
    pltpu.sync_copy(pk_hbm.at[pl.ds(wid * CHUNKS_PER_TILE, CHUNKS_PER_TILE)], pk_v)
    _unpack_edges(pk_v, src_v, dst_v, CHUNKS_PER_TILE)
    rb = s * ROWS_PER_TILE
    _agg_pass(g_hbm, out_hbm, z_hbm, src_v, dst_v,
              (buf_a, buf_b), (sem_a, sem_b), acc, c, rb)


_R = 1024
_GRID = N_PAD // _R


def _pack_bf16_pairs(x):
    u = lax.bitcast_convert_type(x, jnp.int32)
    r = u + 0x7FFF + (lax.shift_right_logical(u, 16) & 1)
    r_lo = r[:, :DW]
    r_hi = r[:, DW:]
    return lax.shift_right_logical(r_lo, 16) | (r_hi & HI_MASK)


def _unpack_bf16_pairs(w):
    lo = lax.bitcast_convert_type(lax.shift_left(w, 16), jnp.float32)
    hi = lax.bitcast_convert_type(w & HI_MASK, jnp.float32)
    return jnp.concatenate([lo, hi], axis=1)


def _tc1_body(x_ref, fi_ref, w1_ref, d0_ref, d1_ref, pa_ref, pb_ref, s_ref):
    deg = d0_ref[...] + d1_ref[...] + 1.0
    s = lax.rsqrt(deg)
    xw = x_ref[...] * jax.nn.sigmoid(fi_ref[...])
    h = jnp.dot(xw, w1_ref[...], preferred_element_type=jnp.float32)
    g = h * s
    pa_ref[...] = _pack_bf16_pairs(g[:, :DH])
    pb_ref[...] = _pack_bf16_pairs(g[:, DH:])
    s_ref[...] = s


def _tc1(x_p, fi, w1, d0, d1):
    return pl.pallas_call(
        _tc1_body,
        grid=(_GRID,),
        in_specs=[
            pl.BlockSpec((_R, D), lambda i: (i, 0)),
            pl.BlockSpec((1, D), lambda i: (0, 0)),
            pl.BlockSpec((D, D), lambda i: (0, 0)),
            pl.BlockSpec((_R, 1), lambda i: (i, 0)),
            pl.BlockSpec((_R, 1), lambda i: (i, 0)),
        ],
        out_specs=[
            pl.BlockSpec((_R, DW), lambda i: (i, 0)),
            pl.BlockSpec((_R, DW), lambda i: (i, 0)),
            pl.BlockSpec((_R, 1), lambda i: (i, 0)),
        ],
        out_shape=[
            jax.ShapeDtypeStruct((N_PAD, DW), jnp.int32),
            jax.ShapeDtypeStruct((N_PAD, DW), jnp.int32),
            jax.ShapeDtypeStruct((N_PAD, 1), jnp.float32),
        ],
    )(x_p, fi, w1, d0, d1)


def _tc2_body(aa_ref, ab_ref, pa_ref, pb_ref, s_ref,
              bias1_ref, w2_ref, g2_ref):
    s = s_ref[...]
    ga = _unpack_bf16_pairs(pa_ref[...])
    gb = _unpack_bf16_pairs(pb_ref[...])
    za = s * (aa_ref[...] + ga) + bias1_ref[:, :DH]
    zb = s * (ab_ref[...] + gb) + bias1_ref[:, DH:]
    ha = jnp.maximum(za, 0.0)
    hb = jnp.maximum(zb, 0.0)
    p = (jnp.dot(ha, w2_ref[:DH, :], preferred_element_type=jnp.float32)
         + jnp.dot(hb, w2_ref[DH:, :], preferred_element_type=jnp.float32))
    g2_ref[...] = p * s


def _tc2(aa, ab, pa, pb, s, bias1, w2p):
    half = pl.BlockSpec((_R, DH), lambda i: (i, 0))
    packed = pl.BlockSpec((_R, DW), lambda i: (i, 0))
    return pl.pallas_call(
        _tc2_body,
        grid=(_GRID,),
        in_specs=[
            half, half, packed, packed,
            pl.BlockSpec((_R, 1), lambda i: (i, 0)),
            pl.BlockSpec((1, D), lambda i: (0, 0)),
            pl.BlockSpec((D, DO), lambda i: (0, 0)),
        ],
        out_specs=pl.BlockSpec((_R, DO), lambda i: (i, 0)),
        out_shape=jax.ShapeDtypeStruct((N_PAD, DO), jnp.float32),
    )(aa, ab, pa, pb, s, bias1, w2p)


def _tc3_body(q0_ref, q1_ref, g2_ref, s_ref, b2_ref, o_ref):
    o_ref[...] = s_ref[...] * (q0_ref[...] + q1_ref[...] + g2_ref[...]) + b2_ref[...]


def _tc3(q0, q1, g2, s, b2p):
    return pl.pallas_call(
        _tc3_body,
        grid=(_GRID,),
        in_specs=[
            pl.BlockSpec((_R, DO), lambda i: (i, 0)),
            pl.BlockSpec((_R, DO), lambda i: (i, 0)),
            pl.BlockSpec((_R, DO), lambda i: (i, 0)),
            pl.BlockSpec((_R, 1), lambda i: (i, 0)),
            pl.BlockSpec((1, DO), lambda i: (0, 0)),
        ],
        out_specs=pl.BlockSpec((_R, DO), lambda i: (i, 0)),
        out_shape=jax.ShapeDtypeStruct((N_PAD, DO), jnp.float32),
    )(q0, q1, g2, s, b2p)


def kernel(x, edge_index, feature_importance, W1, b1, W2, b2):
    f32 = jnp.float32
    pad_e = E_PAD - E
    src = jnp.concatenate([edge_index[0], jnp.full((pad_e,), N, jnp.int32)])
    dst = jnp.concatenate([edge_index[1], jnp.full((pad_e,), N, jnp.int32)])
    packed = ((dst << SHIFT) | src).reshape(ROW_BLOCKS, CHUNK)

    x_p = jnp.zeros((N_PAD, D), f32).at[:N].set(x)
    fi = feature_importance.reshape(1, D)
    b1r = b1.reshape(1, D)
    w2p = jnp.zeros((D, DO), f32).at[:, : W2.shape[1]].set(W2)
    b2p = jnp.zeros((1, DO), f32).at[0, : b2.shape[0]].set(b2)

    ones_c = jnp.ones((CHUNK,), f32)
    z1 = jnp.zeros((ROWS_PER_TILE,), f32)
    z_half = jnp.zeros((CHUNK, DH), f32)
    z_narrow = jnp.zeros((CHUNK, DO), f32)

    deg_parts = _sc_degree(packed, ones_c, z1).reshape(2, N_PAD)
    d0 = deg_parts[0].reshape(N_PAD, 1)
    d1 = deg_parts[1].reshape(N_PAD, 1)

    pa, pb, s = _tc1(x_p, fi, W1, d0, d1)
    g_stack = jnp.stack([pa, pb])

    agg1 = _sc_agg_wide(g_stack, packed, z_half)

    g2 = _tc2(agg1[0], agg1[1], pa, pb, s, b1r, w2p)

    agg2 = _sc_agg_narrow(g2, packed, z_narrow)

    out = _tc3(agg2[0], agg2[1], g2, s, b2p)
    return out[:N, : W2.shape[1]]
